# baseline (speedup 1.0000x reference)
import functools

import jax
import jax.numpy as jnp
from jax import lax
from jax.experimental import pallas as pl
from jax.experimental.pallas import tpu as pltpu
from jax.experimental.pallas import tpu_sc as plsc

VOCAB = 50304
D = 1024
N = 4 * 2048

_info = plsc.get_sparse_core_info()
NC, NS = _info.num_cores, _info.num_subcores
NW = NC * NS
PER_W = N // NW
CHUNK = 32
NCHUNK = PER_W // CHUNK

NBUF = 3

_mesh = plsc.VectorSubcoreMesh(core_axis_name="c", subcore_axis_name="s")


@functools.partial(
    pl.kernel,
    out_type=jax.ShapeDtypeStruct((N, D), jnp.float32),
    mesh=_mesh,
    scratch_types=[
        pltpu.VMEM((NCHUNK, CHUNK), jnp.int32),
        tuple(pltpu.VMEM((CHUNK, D), jnp.float32) for _ in range(NBUF)),
        tuple(pltpu.SemaphoreType.DMA for _ in range(NBUF)),
        tuple(pltpu.SemaphoreType.DMA for _ in range(NBUF)),
    ],
)
def _gather1(idx_hbm, w, o, idx_v, bufs, gsems, wsems):
    wid = lax.axis_index("s") * NC + lax.axis_index("c")
    base = wid * PER_W
    for c in range(NCHUNK):
        pltpu.sync_copy(idx_hbm.at[pl.ds(base + c * CHUNK, CHUNK)],
                        idx_v.at[c])
    gdesc = [None] * NBUF
    wdesc = [None] * NBUF
    for i in range(NCHUNK + 1):
        if i >= 1:
            k = i - 1
            s = k % NBUF
            gdesc[s].wait()
            wdesc[s] = pltpu.async_copy(
                bufs[s], o.at[pl.ds(base + k * CHUNK, CHUNK)], wsems[s])
        if i < NCHUNK:
            s = i % NBUF
            if wdesc[s] is not None:
                wdesc[s].wait()
            gdesc[s] = pltpu.async_copy(
                w.at[idx_v.at[i]], bufs[s], gsems[s])
    for d in wdesc:
        if d is not None:
            d.wait()


def _dup_body(src, dst, sem):
    cp = pltpu.make_async_copy(src, dst, sem)
    cp.start()
    cp.wait()


_dup = pl.pallas_call(
    _dup_body,
    out_shape=jax.ShapeDtypeStruct((N, D), jnp.float32),
    in_specs=[pl.BlockSpec(memory_space=pl.ANY)],
    out_specs=pl.BlockSpec(memory_space=pl.ANY),
    scratch_shapes=[pltpu.SemaphoreType.DMA],
)


def kernel(inputs, W0, W1, W2, W3, W4, W5):
    B, S = inputs.shape
    flat = inputs.reshape(N)
    ve = [_gather1(flat, W) for W in (W0, W1, W2, W3, W4, W5)]
    dups = [_dup(o) for o in ve]
    outs = ve + dups[::-1]
    return tuple(o.reshape(B, S, D) for o in outs)


# speedup vs baseline: 19.9259x; 19.9259x over previous
import functools

import jax
import jax.numpy as jnp
from jax import lax
from jax.experimental import pallas as pl
from jax.experimental.pallas import tpu as pltpu
from jax.experimental.pallas import tpu_sc as plsc

VOCAB = 50304
D = 1024
N = 4 * 2048

_info = plsc.get_sparse_core_info()
NC, NS = _info.num_cores, _info.num_subcores
NW = NC * NS
PER_W = N // NW
CHUNK = 32
NCHUNK = PER_W // CHUNK

NBUF = 3

_mesh = plsc.VectorSubcoreMesh(core_axis_name="c", subcore_axis_name="s")


@functools.partial(
    pl.kernel,
    out_type=jax.ShapeDtypeStruct((N, D), jnp.float32),
    mesh=_mesh,
    scratch_types=[
        pltpu.VMEM((NCHUNK, CHUNK), jnp.int32),
        tuple(pltpu.VMEM((CHUNK, D), jnp.float32) for _ in range(NBUF)),
        tuple(pltpu.SemaphoreType.DMA for _ in range(NBUF)),
        tuple(pltpu.SemaphoreType.DMA for _ in range(NBUF)),
    ],
)
def _gather1(idx_hbm, w, o, idx_v, bufs, gsems, wsems):
    wid = lax.axis_index("s") * NC + lax.axis_index("c")
    base = wid * PER_W
    for c in range(NCHUNK):
        pltpu.sync_copy(idx_hbm.at[pl.ds(base + c * CHUNK, CHUNK)],
                        idx_v.at[c])
    gdesc = [None] * NBUF
    wdesc = [None] * NBUF
    for i in range(NCHUNK + 1):
        if i >= 1:
            k = i - 1
            s = k % NBUF
            gdesc[s].wait()
            wdesc[s] = pltpu.async_copy(
                bufs[s], o.at[pl.ds(base + k * CHUNK, CHUNK)], wsems[s])
        if i < NCHUNK:
            s = i % NBUF
            if wdesc[s] is not None:
                wdesc[s].wait()
            gdesc[s] = pltpu.async_copy(
                w.at[idx_v.at[i]], bufs[s], gsems[s])
    for d in wdesc:
        if d is not None:
            d.wait()


_DUP_ROWS = 512


def _dup_body(src, dst):
    dst[...] = src[...]


_dup = pl.pallas_call(
    _dup_body,
    grid=(N // _DUP_ROWS,),
    in_specs=[pl.BlockSpec((_DUP_ROWS, D), lambda i: (i, 0))],
    out_specs=pl.BlockSpec((_DUP_ROWS, D), lambda i: (i, 0)),
    out_shape=jax.ShapeDtypeStruct((N, D), jnp.float32),
)


def kernel(inputs, W0, W1, W2, W3, W4, W5):
    B, S = inputs.shape
    flat = inputs.reshape(N)
    ve = [_gather1(flat, W) for W in (W0, W1, W2, W3, W4, W5)]
    dups = [_dup(o) for o in ve]
    outs = ve + dups[::-1]
    return tuple(o.reshape(B, S, D) for o in outs)


# speedup vs baseline: 23.4668x; 1.1777x over previous
import functools

import jax
import jax.numpy as jnp
from jax import lax
from jax.experimental import pallas as pl
from jax.experimental.pallas import tpu as pltpu
from jax.experimental.pallas import tpu_sc as plsc

VOCAB = 50304
D = 1024
N = 4 * 2048

_info = plsc.get_sparse_core_info()
NC, NS = _info.num_cores, _info.num_subcores
NW = NC * NS
PER_W = N // NW
CHUNK = 32
NCHUNK = PER_W // CHUNK

NBUF = 3

_mesh = plsc.VectorSubcoreMesh(core_axis_name="c", subcore_axis_name="s")


def _gather_body(idx_hbm, tables, prim, dup, idx_v, bufs, gsems, wsems):
    wid = lax.axis_index("s") * NC + lax.axis_index("c")
    base = wid * PER_W
    for c in range(NCHUNK):
        pltpu.sync_copy(idx_hbm.at[pl.ds(base + c * CHUNK, CHUNK)],
                        idx_v.at[c])
    tasks = [(w, prim[t], dup[t] if dup is not None else None, c)
             for t, w in enumerate(tables)
             for c in range(NCHUNK)]
    nt = len(tasks)
    gdesc = [None] * NBUF
    wdesc = [None] * NBUF
    for i in range(nt + 1):
        if i >= 1:
            k = i - 1
            s = k % NBUF
            _, oa, ob, c = tasks[k]
            gdesc[s].wait()
            dst = pl.ds(base + c * CHUNK, CHUNK)
            ds_ = [pltpu.async_copy(bufs[s], oa.at[dst], wsems[s])]
            if ob is not None:
                ds_.append(pltpu.async_copy(bufs[s], ob.at[dst], wsems[s]))
            wdesc[s] = ds_
        if i < nt:
            s = i % NBUF
            w, _, _, c = tasks[i]
            if wdesc[s] is not None:
                for d in wdesc[s]:
                    d.wait()
            gdesc[s] = pltpu.async_copy(
                w.at[idx_v.at[c]], bufs[s], gsems[s])
    for ds_ in wdesc:
        if ds_ is not None:
            for d in ds_:
                d.wait()


_SCRATCH = [
    pltpu.VMEM((NCHUNK, CHUNK), jnp.int32),
    tuple(pltpu.VMEM((CHUNK, D), jnp.float32) for _ in range(NBUF)),
    tuple(pltpu.SemaphoreType.DMA for _ in range(NBUF)),
    tuple(pltpu.SemaphoreType.DMA for _ in range(NBUF)),
]
_OUT = jax.ShapeDtypeStruct((N, D), jnp.float32)


@functools.partial(pl.kernel, out_type=(_OUT,) * 2, mesh=_mesh,
                   scratch_types=_SCRATCH)
def _gather_a(idx_hbm, w0, w1, o0, o1, idx_v, bufs, gsems, wsems):
    _gather_body(idx_hbm, (w0, w1), (o0, o1), None,
                 idx_v, bufs, gsems, wsems)


@functools.partial(pl.kernel, out_type=(_OUT,) * 8, mesh=_mesh,
                   scratch_types=_SCRATCH)
def _gather_b(idx_hbm, w2, w3, w4, w5,
              o2, o3, o4, o5, d2, d3, d4, d5,
              idx_v, bufs, gsems, wsems):
    _gather_body(idx_hbm, (w2, w3, w4, w5), (o2, o3, o4, o5),
                 (d2, d3, d4, d5), idx_v, bufs, gsems, wsems)


_DUP_ROWS = 512


def _dup_body(src, dst):
    dst[...] = src[...]


_dup = pl.pallas_call(
    _dup_body,
    grid=(N // _DUP_ROWS,),
    in_specs=[pl.BlockSpec((_DUP_ROWS, D), lambda i: (i, 0))],
    out_specs=pl.BlockSpec((_DUP_ROWS, D), lambda i: (i, 0)),
    out_shape=jax.ShapeDtypeStruct((N, D), jnp.float32),
)


def kernel(inputs, W0, W1, W2, W3, W4, W5):
    B, S = inputs.shape
    flat = inputs.reshape(N)
    o0, o1 = _gather_a(flat, W0, W1)
    o2, o3, o4, o5, d2, d3, d4, d5 = _gather_b(flat, W2, W3, W4, W5)
    d0 = _dup(o0)
    d1 = _dup(o1)
    outs = [o0, o1, o2, o3, o4, o5, d5, d4, d3, d2, d1, d0]
    return tuple(o.reshape(B, S, D) for o in outs)
